# baseline (speedup 1.0000x reference)
import jax
import jax.numpy as jnp
from jax.experimental import pallas as pl
from jax.experimental.pallas import tpu as pltpu

N = 100000
F = 128
G = 64
EPS = 1e-05
BLK = 2000
NBLK = N // BLK
_PREC = jax.lax.Precision.HIGHEST


def _body(x_ref, b_ref, w_ref, bias_ref, ms_ref, y_ref,
          stash, sums, sq, cnt, atab, btab):
    p = pl.program_id(0)
    j = pl.program_id(1)
    b = b_ref[0, 0, :]
    oh = (b[:, None] == jax.lax.broadcasted_iota(jnp.int32, (BLK, G), 1)).astype(
        jnp.float32
    )

    @pl.when(p == 0)
    def _stats():
        x = x_ref[...]
        stash[pl.ds(j * BLK, BLK), :] = x
        s = jax.lax.dot_general(oh, x, (((0,), (0,)), ((), ())), precision=_PREC)
        q = jax.lax.dot_general(oh, x * x, (((0,), (0,)), ((), ())),
                                precision=_PREC)
        c = jnp.broadcast_to(jnp.sum(oh, axis=0)[None, :], (8, G))

        @pl.when(j == 0)
        def _init():
            sums[...] = s
            sq[...] = q
            cnt[...] = c

        @pl.when(j != 0)
        def _acc():
            sums[...] += s
            sq[...] += q
            cnt[...] += c

    @pl.when(p == 1)
    def _apply():
        @pl.when(j == 0)
        def _tables():
            inv_c = 1.0 / jnp.maximum(cnt[0, :], 1.0)[:, None]
            m = sums[...] * inv_c
            qm = sq[...] * inv_c
            ms = ms_ref[0, :][None, :]
            var = qm - m * m * ms * (2.0 - ms)
            a = w_ref[0, :][None, :] * jax.lax.rsqrt(var + EPS)
            atab[...] = a
            btab[...] = bias_ref[0, :][None, :] - a * m * ms

        arow = jax.lax.dot_general(oh, atab[...], (((1,), (0,)), ((), ())),
                                   precision=_PREC)
        brow = jax.lax.dot_general(oh, btab[...], (((1,), (0,)), ((), ())),
                                   precision=_PREC)
        y_ref[...] = arow * stash[pl.ds(j * BLK, BLK), :] + brow


@jax.jit
def kernel(x, batch, weight, bias, mean_scale):
    b3 = batch.astype(jnp.int32).reshape(NBLK, 1, BLK)
    w2 = weight.reshape(1, F)
    bias2 = bias.reshape(1, F)
    ms2 = mean_scale.reshape(1, F)

    return pl.pallas_call(
        _body,
        grid=(2, NBLK),
        in_specs=[
            pl.BlockSpec((BLK, F), lambda p, j: (jnp.where(p == 0, j, 0), 0)),
            pl.BlockSpec((1, 1, BLK), lambda p, j: (j, 0, 0)),
            pl.BlockSpec((1, F), lambda p, j: (0, 0)),
            pl.BlockSpec((1, F), lambda p, j: (0, 0)),
            pl.BlockSpec((1, F), lambda p, j: (0, 0)),
        ],
        out_specs=pl.BlockSpec((BLK, F), lambda p, j: (jnp.where(p == 0, 0, j), 0)),
        out_shape=jax.ShapeDtypeStruct((N, F), jnp.float32),
        scratch_shapes=[
            pltpu.VMEM((N, F), jnp.float32),
            pltpu.VMEM((G, F), jnp.float32),
            pltpu.VMEM((G, F), jnp.float32),
            pltpu.VMEM((8, G), jnp.float32),
            pltpu.VMEM((G, F), jnp.float32),
            pltpu.VMEM((G, F), jnp.float32),
        ],
    )(x, b3, w2, bias2, ms2)


# speedup vs baseline: 1.9947x; 1.9947x over previous
import jax
import jax.numpy as jnp
from jax.experimental import pallas as pl
from jax.experimental.pallas import tpu as pltpu

N = 100000
F = 128
G = 64
EPS = 1e-05
BLK = 2000
NBLK = N // BLK
_PREC = jax.lax.Precision.HIGHEST


def _body(x_ref, b_ref, w_ref, bias_ref, ms_ref, y_ref,
          stash, sumsq, cnt, ab):
    p = pl.program_id(0)
    j = pl.program_id(1)
    b = b_ref[0, 0, :]
    oh = (b[:, None] == jax.lax.broadcasted_iota(jnp.int32, (BLK, G), 1)).astype(
        jnp.bfloat16
    )

    @pl.when(p == 0)
    def _stats():
        x = x_ref[...]
        stash[pl.ds(j * BLK, BLK), :] = x
        xq = jnp.concatenate([x, x * x], axis=1).astype(jnp.bfloat16)
        s = jax.lax.dot_general(oh, xq, (((0,), (0,)), ((), ())),
                                preferred_element_type=jnp.float32)
        c = jnp.broadcast_to(
            jnp.sum(oh.astype(jnp.float32), axis=0)[None, :], (8, G))

        @pl.when(j == 0)
        def _init():
            sumsq[...] = s
            cnt[...] = c

        @pl.when(j != 0)
        def _acc():
            sumsq[...] += s
            cnt[...] += c

    @pl.when(p == 1)
    def _apply():
        @pl.when(j == 0)
        def _tables():
            inv_c = 1.0 / jnp.maximum(cnt[0, :], 1.0)[:, None]
            m = sumsq[:, :F] * inv_c
            qm = sumsq[:, F:] * inv_c
            ms = ms_ref[0, :][None, :]
            var = qm - m * m * ms * (2.0 - ms)
            a = w_ref[0, :][None, :] * jax.lax.rsqrt(var + EPS)
            ab[:, :F] = a.astype(jnp.bfloat16)
            ab[:, F:] = (bias_ref[0, :][None, :] - a * m * ms).astype(jnp.bfloat16)

        abrow = jax.lax.dot_general(oh, ab[...], (((1,), (0,)), ((), ())),
                                    preferred_element_type=jnp.float32)
        y_ref[...] = abrow[:, :F] * stash[pl.ds(j * BLK, BLK), :] + abrow[:, F:]


@jax.jit
def kernel(x, batch, weight, bias, mean_scale):
    b3 = batch.astype(jnp.int32).reshape(NBLK, 1, BLK)
    w2 = weight.reshape(1, F)
    bias2 = bias.reshape(1, F)
    ms2 = mean_scale.reshape(1, F)

    return pl.pallas_call(
        _body,
        grid=(2, NBLK),
        in_specs=[
            pl.BlockSpec((BLK, F), lambda p, j: (jnp.where(p == 0, j, 0), 0)),
            pl.BlockSpec((1, 1, BLK), lambda p, j: (j, 0, 0)),
            pl.BlockSpec((1, F), lambda p, j: (0, 0)),
            pl.BlockSpec((1, F), lambda p, j: (0, 0)),
            pl.BlockSpec((1, F), lambda p, j: (0, 0)),
        ],
        out_specs=pl.BlockSpec((BLK, F), lambda p, j: (jnp.where(p == 0, 0, j), 0)),
        out_shape=jax.ShapeDtypeStruct((N, F), jnp.float32),
        scratch_shapes=[
            pltpu.VMEM((N, F), jnp.float32),
            pltpu.VMEM((G, 2 * F), jnp.float32),
            pltpu.VMEM((8, G), jnp.float32),
            pltpu.VMEM((G, 2 * F), jnp.bfloat16),
        ],
    )(x, b3, w2, bias2, ms2)


# speedup vs baseline: 2.0048x; 1.0051x over previous
import jax
import jax.numpy as jnp
from jax.experimental import pallas as pl
from jax.experimental.pallas import tpu as pltpu

N = 100000
F = 128
G = 64
EPS = 1e-05
BLK = 2000
NBLK = N // BLK
_PREC = jax.lax.Precision.HIGHEST


def _body(x_ref, b_ref, w_ref, bias_ref, ms_ref, y_ref,
          stash, sumsq, cnt, ab):
    p = pl.program_id(0)
    j = pl.program_id(1)
    b = b_ref[0, 0, :]
    oh = (b[:, None] == jax.lax.broadcasted_iota(jnp.int32, (BLK, G), 1)).astype(
        jnp.bfloat16
    )

    @pl.when(p == 0)
    def _stats():
        x = x_ref[...]
        stash[pl.ds(j * BLK, BLK), :] = x
        xb = x.astype(jnp.bfloat16)
        xq = jnp.concatenate([xb, xb * xb], axis=1)
        s = jax.lax.dot_general(oh, xq, (((0,), (0,)), ((), ())),
                                preferred_element_type=jnp.float32)
        ones8 = jnp.ones((8, BLK), dtype=jnp.bfloat16)
        c = jax.lax.dot_general(ones8, oh, (((1,), (0,)), ((), ())),
                                preferred_element_type=jnp.float32)

        @pl.when(j == 0)
        def _init():
            sumsq[...] = s
            cnt[...] = c

        @pl.when(j != 0)
        def _acc():
            sumsq[...] += s
            cnt[...] += c

    @pl.when(p == 1)
    def _apply():
        @pl.when(j == 0)
        def _tables():
            inv_c = 1.0 / jnp.maximum(cnt[0, :], 1.0)[:, None]
            m = sumsq[:, :F] * inv_c
            qm = sumsq[:, F:] * inv_c
            ms = ms_ref[0, :][None, :]
            var = qm - m * m * ms * (2.0 - ms)
            a = w_ref[0, :][None, :] * jax.lax.rsqrt(var + EPS)
            ab[:, :F] = a.astype(jnp.bfloat16)
            ab[:, F:] = (bias_ref[0, :][None, :] - a * m * ms).astype(jnp.bfloat16)

        abrow = jax.lax.dot_general(oh, ab[...], (((1,), (0,)), ((), ())),
                                    preferred_element_type=jnp.float32)
        y_ref[...] = abrow[:, :F] * stash[pl.ds(j * BLK, BLK), :] + abrow[:, F:]


@jax.jit
def kernel(x, batch, weight, bias, mean_scale):
    b3 = batch.astype(jnp.int32).reshape(NBLK, 1, BLK)
    w2 = weight.reshape(1, F)
    bias2 = bias.reshape(1, F)
    ms2 = mean_scale.reshape(1, F)

    return pl.pallas_call(
        _body,
        grid=(2, NBLK),
        in_specs=[
            pl.BlockSpec((BLK, F), lambda p, j: (jnp.where(p == 0, j, 0), 0)),
            pl.BlockSpec((1, 1, BLK), lambda p, j: (j, 0, 0)),
            pl.BlockSpec((1, F), lambda p, j: (0, 0)),
            pl.BlockSpec((1, F), lambda p, j: (0, 0)),
            pl.BlockSpec((1, F), lambda p, j: (0, 0)),
        ],
        out_specs=pl.BlockSpec((BLK, F), lambda p, j: (jnp.where(p == 0, 0, j), 0)),
        out_shape=jax.ShapeDtypeStruct((N, F), jnp.float32),
        scratch_shapes=[
            pltpu.VMEM((N, F), jnp.float32),
            pltpu.VMEM((G, 2 * F), jnp.float32),
            pltpu.VMEM((8, G), jnp.float32),
            pltpu.VMEM((G, 2 * F), jnp.bfloat16),
        ],
    )(x, b3, w2, bias2, ms2)


# speedup vs baseline: 2.8638x; 1.4285x over previous
import jax
import jax.numpy as jnp
from jax.experimental import pallas as pl
from jax.experimental.pallas import tpu as pltpu

N = 100000
F = 128
G = 64
EPS = 1e-05
BLK = 5000
NBLK = N // BLK
_PREC = jax.lax.Precision.HIGHEST


def _body(x_ref, b_ref, w_ref, bias_ref, ms_ref, y_ref,
          stash, sumsq, cnt, ab):
    p = pl.program_id(0)
    j = pl.program_id(1)
    b = b_ref[0, 0, :]
    oh = (b[:, None] == jax.lax.broadcasted_iota(jnp.int32, (BLK, G), 1)).astype(
        jnp.bfloat16
    )

    @pl.when(p == 0)
    def _stats():
        x = x_ref[...]
        xb = x.astype(jnp.bfloat16)
        stash[pl.ds(j * BLK, BLK), :] = xb
        xq = jnp.concatenate([xb, xb * xb], axis=1)
        s = jax.lax.dot_general(oh, xq, (((0,), (0,)), ((), ())),
                                preferred_element_type=jnp.float32)
        ones8 = jnp.ones((8, BLK), dtype=jnp.bfloat16)
        c = jax.lax.dot_general(ones8, oh, (((1,), (0,)), ((), ())),
                                preferred_element_type=jnp.float32)

        @pl.when(j == 0)
        def _init():
            sumsq[...] = s
            cnt[...] = c

        @pl.when(j != 0)
        def _acc():
            sumsq[...] += s
            cnt[...] += c

    @pl.when(p == 1)
    def _apply():
        @pl.when(j == 0)
        def _tables():
            inv_c = 1.0 / jnp.maximum(cnt[0, :], 1.0)[:, None]
            m = sumsq[:, :F] * inv_c
            qm = sumsq[:, F:] * inv_c
            ms = ms_ref[0, :][None, :]
            var = qm - m * m * ms * (2.0 - ms)
            a = w_ref[0, :][None, :] * jax.lax.rsqrt(var + EPS)
            ab[:, :F] = a.astype(jnp.bfloat16)
            ab[:, F:] = (bias_ref[0, :][None, :] - a * m * ms).astype(jnp.bfloat16)

        abrow = jax.lax.dot_general(oh, ab[...], (((1,), (0,)), ((), ())),
                                    preferred_element_type=jnp.float32)
        xs = stash[pl.ds(j * BLK, BLK), :].astype(jnp.float32)
        y_ref[...] = abrow[:, :F] * xs + abrow[:, F:]


@jax.jit
def kernel(x, batch, weight, bias, mean_scale):
    b3 = batch.astype(jnp.int32).reshape(NBLK, 1, BLK)
    w2 = weight.reshape(1, F)
    bias2 = bias.reshape(1, F)
    ms2 = mean_scale.reshape(1, F)

    return pl.pallas_call(
        _body,
        grid=(2, NBLK),
        in_specs=[
            pl.BlockSpec((BLK, F), lambda p, j: (jnp.where(p == 0, j, 0), 0)),
            pl.BlockSpec((1, 1, BLK), lambda p, j: (j, 0, 0)),
            pl.BlockSpec((1, F), lambda p, j: (0, 0)),
            pl.BlockSpec((1, F), lambda p, j: (0, 0)),
            pl.BlockSpec((1, F), lambda p, j: (0, 0)),
        ],
        out_specs=pl.BlockSpec((BLK, F), lambda p, j: (jnp.where(p == 0, 0, j), 0)),
        out_shape=jax.ShapeDtypeStruct((N, F), jnp.float32),
        scratch_shapes=[
            pltpu.VMEM((N, F), jnp.bfloat16),
            pltpu.VMEM((G, 2 * F), jnp.float32),
            pltpu.VMEM((8, G), jnp.float32),
            pltpu.VMEM((G, 2 * F), jnp.bfloat16),
        ],
    )(x, b3, w2, bias2, ms2)


# speedup vs baseline: 3.1233x; 1.0906x over previous
import jax
import jax.numpy as jnp
from jax.experimental import pallas as pl
from jax.experimental.pallas import tpu as pltpu

N = 100000
F = 128
G = 64
EPS = 1e-05
BLK = 10000
NBLK = N // BLK
_PREC = jax.lax.Precision.HIGHEST


def _body(x_ref, b_ref, w_ref, bias_ref, ms_ref, y_ref,
          stash, sumsq, cnt, ab):
    p = pl.program_id(0)
    j = pl.program_id(1)
    b = b_ref[0, 0, :]
    oh = (b[:, None] == jax.lax.broadcasted_iota(jnp.int32, (BLK, G), 1)).astype(
        jnp.bfloat16
    )

    @pl.when(p == 0)
    def _stats():
        x = x_ref[...]
        xb = x.astype(jnp.bfloat16)
        stash[pl.ds(j * BLK, BLK), :] = xb
        xq = jnp.concatenate([xb, xb * xb], axis=1)
        s = jax.lax.dot_general(oh, xq, (((0,), (0,)), ((), ())),
                                preferred_element_type=jnp.float32)
        ones8 = jnp.ones((8, BLK), dtype=jnp.bfloat16)
        c = jax.lax.dot_general(ones8, oh, (((1,), (0,)), ((), ())),
                                preferred_element_type=jnp.float32)

        @pl.when(j == 0)
        def _init():
            sumsq[...] = s
            cnt[...] = c

        @pl.when(j != 0)
        def _acc():
            sumsq[...] += s
            cnt[...] += c

    @pl.when(p == 1)
    def _apply():
        @pl.when(j == 0)
        def _tables():
            inv_c = 1.0 / jnp.maximum(cnt[0, :], 1.0)[:, None]
            m = sumsq[:, :F] * inv_c
            qm = sumsq[:, F:] * inv_c
            ms = ms_ref[0, :][None, :]
            var = qm - m * m * ms * (2.0 - ms)
            a = w_ref[0, :][None, :] * jax.lax.rsqrt(var + EPS)
            ab[:, :F] = a.astype(jnp.bfloat16)
            ab[:, F:] = (bias_ref[0, :][None, :] - a * m * ms).astype(jnp.bfloat16)

        abrow = jax.lax.dot_general(oh, ab[...], (((1,), (0,)), ((), ())),
                                    preferred_element_type=jnp.float32)
        xs = stash[pl.ds(j * BLK, BLK), :].astype(jnp.float32)
        y_ref[...] = abrow[:, :F] * xs + abrow[:, F:]


@jax.jit
def kernel(x, batch, weight, bias, mean_scale):
    b3 = batch.astype(jnp.int32).reshape(NBLK, 1, BLK)
    w2 = weight.reshape(1, F)
    bias2 = bias.reshape(1, F)
    ms2 = mean_scale.reshape(1, F)

    return pl.pallas_call(
        _body,
        grid=(2, NBLK),
        in_specs=[
            pl.BlockSpec((BLK, F), lambda p, j: (jnp.where(p == 0, j, 0), 0)),
            pl.BlockSpec((1, 1, BLK), lambda p, j: (j, 0, 0)),
            pl.BlockSpec((1, F), lambda p, j: (0, 0)),
            pl.BlockSpec((1, F), lambda p, j: (0, 0)),
            pl.BlockSpec((1, F), lambda p, j: (0, 0)),
        ],
        out_specs=pl.BlockSpec((BLK, F), lambda p, j: (jnp.where(p == 0, 0, j), 0)),
        out_shape=jax.ShapeDtypeStruct((N, F), jnp.float32),
        scratch_shapes=[
            pltpu.VMEM((N, F), jnp.bfloat16),
            pltpu.VMEM((G, 2 * F), jnp.float32),
            pltpu.VMEM((8, G), jnp.float32),
            pltpu.VMEM((G, 2 * F), jnp.bfloat16),
        ],
    )(x, b3, w2, bias2, ms2)
